# baseline (speedup 1.0000x reference)
import functools

import jax
import jax.numpy as jnp
from jax import lax
from jax.experimental import pallas as pl
from jax.experimental.pallas import tpu as pltpu
from jax.experimental.pallas import tpu_sc as plsc

N_NODES = 10000
N_GRAPHS = 16
NC = 2
NS = 16
CHUNK = 384
DEG_CHUNK = 128
N_PAD = 10112
ROWS_PER_SUB = N_PAD // NS

_mesh = functools.partial(
    plsc.VectorSubcoreMesh, core_axis_name="c", subcore_axis_name="s")


def _degree_sc(cols, ones128, zeros128):
  cols = cols.reshape(NC, NS, -1, DEG_CHUNK)
  cpt = cols.shape[2]

  @functools.partial(
      pl.kernel,
      out_type=jax.ShapeDtypeStruct((NC, N_PAD, 128), jnp.float32),
      mesh=_mesh(),
      scratch_types=[
          pltpu.VMEM((DEG_CHUNK,), jnp.int32),
          pltpu.VMEM((DEG_CHUNK, 128), jnp.float32),
          pltpu.VMEM_SHARED((N_PAD, 128), jnp.float32),
      ],
  )
  def deg_kernel(cols_h, ones_h, zeros_h, out_h, ci, ones_v, acc):
    cid = lax.axis_index("c")
    sid = lax.axis_index("s")
    r0 = sid * ROWS_PER_SUB
    pltpu.sync_copy(zeros_h.at[pl.ds(r0, ROWS_PER_SUB)],
                    acc.at[pl.ds(r0, ROWS_PER_SUB)])
    pltpu.sync_copy(ones_h, ones_v)
    plsc.subcore_barrier()

    @pl.loop(0, cpt)
    def _(j):
      pltpu.sync_copy(cols_h.at[cid, sid, j], ci)
      pltpu.sync_copy(ones_v, acc.at[ci], add=True)

    plsc.subcore_barrier()
    pltpu.sync_copy(acc.at[pl.ds(r0, ROWS_PER_SUB)],
                    out_h.at[cid, pl.ds(r0, ROWS_PER_SUB)])

  return deg_kernel(cols, ones128, zeros128)


def _aggregate_sc(rows, cols, y, zeros):
  d = y.shape[1]
  cpt = rows.shape[2]

  @functools.partial(
      pl.kernel,
      out_type=jax.ShapeDtypeStruct((NC, N_PAD, d), jnp.float32),
      mesh=_mesh(),
      scratch_types=[
          pltpu.VMEM((CHUNK,), jnp.int32),
          pltpu.VMEM((CHUNK,), jnp.int32),
          pltpu.VMEM((CHUNK, d), jnp.float32),
          pltpu.VMEM_SHARED((N_PAD, d), jnp.float32),
          pltpu.SemaphoreType.DMA,
      ],
  )
  def agg_kernel(rows_h, cols_h, y_h, zeros_h, out_h, ri, ci, g, acc, sem):
    cid = lax.axis_index("c")
    sid = lax.axis_index("s")
    r0 = sid * ROWS_PER_SUB
    pltpu.sync_copy(zeros_h.at[pl.ds(r0, ROWS_PER_SUB)],
                    acc.at[pl.ds(r0, ROWS_PER_SUB)])
    plsc.subcore_barrier()

    @pl.loop(0, cpt)
    def _(j):
      pltpu.sync_copy(rows_h.at[cid, sid, j], ri)
      pltpu.sync_copy(cols_h.at[cid, sid, j], ci)
      pltpu.async_copy(y_h.at[ri], g, sem).wait()
      pltpu.sync_copy(g, acc.at[ci], add=True)

    plsc.subcore_barrier()
    pltpu.sync_copy(acc.at[pl.ds(r0, ROWS_PER_SUB)],
                    out_h.at[cid, pl.ds(r0, ROWS_PER_SUB)])

  return agg_kernel(rows, cols, y, zeros)


def _dot(a, b):
  return jnp.dot(a, b, preferred_element_type=jnp.float32,
                 precision=lax.Precision.HIGHEST)


def _mm_body(x_ref, w_ref, o_ref):
  o_ref[...] = _dot(x_ref[...], w_ref[...])


def _tc_matmul(x, w):
  return pl.pallas_call(
      _mm_body,
      out_shape=jax.ShapeDtypeStruct((x.shape[0], w.shape[1]), jnp.float32),
  )(x, w)


def _prep_body(degp_ref, xw_ref, y_ref, dinv_ref):
  degp = degp_ref[0] + degp_ref[1]
  deg = degp[:, 0:1] + 1.0
  dinv = lax.rsqrt(deg)
  dinv_ref[...] = dinv
  y_ref[0:N_NODES, :] = dinv[0:N_NODES] * xw_ref[...]
  y_ref[N_NODES:N_PAD, :] = jnp.zeros(
      (N_PAD - N_NODES, xw_ref.shape[1]), jnp.float32)


def _layer_body(p_ref, y_ref, dinv_ref, b_ref, w_ref, o_ref):
  s = p_ref[0] + p_ref[1] + y_ref[...]
  dinv = dinv_ref[...]
  h = jnp.maximum(dinv * s + b_ref[...], 0.0)
  xw = _dot(h, w_ref[...])
  rid = lax.broadcasted_iota(jnp.int32, (N_PAD, 1), 0)
  o_ref[...] = jnp.where(rid < N_NODES, dinv * xw, 0.0)


def _final_body(p_ref, y_ref, dinv_ref, b_ref, batch_ref,
                wf1_ref, bf1_ref, wf2_ref, bf2_ref, o_ref):
  s = p_ref[0] + p_ref[1] + y_ref[...]
  h = jnp.maximum(dinv_ref[...] * s + b_ref[...], 0.0)
  hn = h[0:N_NODES, 0:64]
  gid = lax.broadcasted_iota(jnp.int32, (N_GRAPHS, N_NODES), 0)
  mask = (batch_ref[...] == gid).astype(jnp.float32)
  sums = _dot(mask, hn)
  cnt = jnp.sum(mask, axis=1, keepdims=True)
  pooled = sums / jnp.maximum(cnt, 1.0)
  hf = jnp.maximum(_dot(pooled, wf1_ref[...]) + bf1_ref[...], 0.0)
  o_ref[...] = _dot(hf, wf2_ref[...]) + bf2_ref[...]


def kernel(x, edge_index, batch, W1, b1, W2, b2, W3, b3, Wf1, bf1, Wf2, bf2):
  e = edge_index.shape[1]
  per_tile = -(-e // (NC * NS))
  cpt = -(-per_tile // CHUNK)
  cpt += cpt % 2
  e_pad = NC * NS * cpt * CHUNK

  row = edge_index[0].astype(jnp.int32)
  col = edge_index[1].astype(jnp.int32)
  fill = jnp.full((e_pad - e,), N_NODES, jnp.int32)
  rows = jnp.concatenate([row, fill]).reshape(NC, NS, cpt, CHUNK)
  cols = jnp.concatenate([col, fill]).reshape(NC, NS, cpt, CHUNK)

  ones128 = jnp.ones((DEG_CHUNK, 128), jnp.float32)
  zeros128 = jnp.zeros((N_PAD, 128), jnp.float32)

  W3p = jnp.pad(W3, ((0, 0), (0, 128 - W3.shape[1])))
  b3p = jnp.pad(b3, (0, 128 - b3.shape[0]))

  degp = _degree_sc(cols, ones128, zeros128)
  xw1 = _tc_matmul(x, W1)

  y1, dinv = pl.pallas_call(
      _prep_body,
      out_shape=(jax.ShapeDtypeStruct((N_PAD, 128), jnp.float32),
                 jax.ShapeDtypeStruct((N_PAD, 1), jnp.float32)),
  )(degp, xw1)

  p1 = _aggregate_sc(rows, cols, y1, zeros128)
  y2 = pl.pallas_call(
      _layer_body,
      out_shape=jax.ShapeDtypeStruct((N_PAD, 128), jnp.float32),
  )(p1, y1, dinv, b1.reshape(1, -1), W2)

  p2 = _aggregate_sc(rows, cols, y2, zeros128)
  y3 = pl.pallas_call(
      _layer_body,
      out_shape=jax.ShapeDtypeStruct((N_PAD, 128), jnp.float32),
  )(p2, y2, dinv, b2.reshape(1, -1), W3p)

  p3 = _aggregate_sc(rows, cols, y3, zeros128)
  out = pl.pallas_call(
      _final_body,
      out_shape=jax.ShapeDtypeStruct((N_GRAPHS, Wf2.shape[1]), jnp.float32),
  )(p3, y3, dinv, b3p.reshape(1, -1),
    batch.astype(jnp.int32).reshape(1, -1),
    Wf1, bf1.reshape(1, -1), Wf2, bf2.reshape(1, -1))
  return out


# speedup vs baseline: 2.0204x; 2.0204x over previous
import functools

import jax
import jax.numpy as jnp
from jax import lax
from jax.experimental import pallas as pl
from jax.experimental.pallas import tpu as pltpu
from jax.experimental.pallas import tpu_sc as plsc

N_NODES = 10000
N_GRAPHS = 16
NC = 2
NS = 16
CHUNK = 256
DEG_CHUNK = 128
N_PAD = 10112
ROWS_PER_SUB = N_PAD // NS

_mesh = functools.partial(
    plsc.VectorSubcoreMesh, core_axis_name="c", subcore_axis_name="s")


def _degree_sc(cols, ones128, zeros128):
  cols = cols.reshape(NC, NS, -1, DEG_CHUNK)
  cpt = cols.shape[2]

  @functools.partial(
      pl.kernel,
      out_type=jax.ShapeDtypeStruct((NC, N_PAD, 128), jnp.float32),
      mesh=_mesh(),
      scratch_types=[
          pltpu.VMEM((DEG_CHUNK,), jnp.int32),
          pltpu.VMEM((DEG_CHUNK, 128), jnp.float32),
          pltpu.VMEM_SHARED((N_PAD, 128), jnp.float32),
      ],
  )
  def deg_kernel(cols_h, ones_h, zeros_h, out_h, ci, ones_v, acc):
    cid = lax.axis_index("c")
    sid = lax.axis_index("s")
    r0 = sid * ROWS_PER_SUB
    pltpu.sync_copy(zeros_h.at[pl.ds(r0, ROWS_PER_SUB)],
                    acc.at[pl.ds(r0, ROWS_PER_SUB)])
    pltpu.sync_copy(ones_h, ones_v)
    plsc.subcore_barrier()

    @pl.loop(0, cpt)
    def _(j):
      pltpu.sync_copy(cols_h.at[cid, sid, j], ci)
      pltpu.sync_copy(ones_v, acc.at[ci], add=True)

    plsc.subcore_barrier()
    pltpu.sync_copy(acc.at[pl.ds(r0, ROWS_PER_SUB)],
                    out_h.at[cid, pl.ds(r0, ROWS_PER_SUB)])

  return deg_kernel(cols, ones128, zeros128)


def _aggregate_sc(rows, cols, y, zeros):
  d = y.shape[1]
  cpt = rows.shape[2]

  @functools.partial(
      pl.kernel,
      out_type=jax.ShapeDtypeStruct((NC, N_PAD, d), jnp.float32),
      mesh=_mesh(),
      scratch_types=[
          pltpu.VMEM((CHUNK,), jnp.int32),
          pltpu.VMEM((CHUNK,), jnp.int32),
          pltpu.VMEM((CHUNK, d), jnp.float32),
          pltpu.VMEM_SHARED((N_PAD, d), jnp.float32),
          pltpu.SemaphoreType.DMA,
      ],
  )
  def agg_kernel(rows_h, cols_h, y_h, zeros_h, out_h, ri, ci, g, acc, sem):
    cid = lax.axis_index("c")
    sid = lax.axis_index("s")
    r0 = sid * ROWS_PER_SUB
    pltpu.sync_copy(zeros_h.at[pl.ds(r0, ROWS_PER_SUB)],
                    acc.at[pl.ds(r0, ROWS_PER_SUB)])
    plsc.subcore_barrier()

    @pl.loop(0, cpt)
    def _(j):
      pltpu.sync_copy(rows_h.at[cid, sid, j], ri)
      pltpu.sync_copy(cols_h.at[cid, sid, j], ci)
      pltpu.async_copy(y_h.at[ri], g, sem).wait()
      pltpu.sync_copy(g, acc.at[ci], add=True)

    plsc.subcore_barrier()
    pltpu.sync_copy(acc.at[pl.ds(r0, ROWS_PER_SUB)],
                    out_h.at[cid, pl.ds(r0, ROWS_PER_SUB)])

  return agg_kernel(rows, cols, y, zeros)


def _dot(a, b):
  return jnp.dot(a, b, preferred_element_type=jnp.float32,
                 precision=lax.Precision.HIGHEST)


def _mm_body(x_ref, w_ref, o_ref):
  o_ref[...] = _dot(x_ref[...], w_ref[...])


def _tc_matmul(x, w):
  return pl.pallas_call(
      _mm_body,
      out_shape=jax.ShapeDtypeStruct((x.shape[0], w.shape[1]), jnp.float32),
  )(x, w)


def _prep_body(degp_ref, xw_ref, y_ref, dinv_ref):
  degp = degp_ref[0] + degp_ref[1]
  deg = degp[:, 0:1] + 1.0
  dinv = lax.rsqrt(deg)
  dinv_ref[...] = dinv
  y_ref[0:N_NODES, :] = dinv[0:N_NODES] * xw_ref[...]
  y_ref[N_NODES:N_PAD, :] = jnp.zeros(
      (N_PAD - N_NODES, xw_ref.shape[1]), jnp.float32)


def _layer_body(p_ref, y_ref, dinv_ref, b_ref, w_ref, o_ref):
  s = p_ref[0] + p_ref[1] + y_ref[...]
  dinv = dinv_ref[...]
  h = jnp.maximum(dinv * s + b_ref[...], 0.0)
  xw = _dot(h, w_ref[...])
  rid = lax.broadcasted_iota(jnp.int32, (N_PAD, 1), 0)
  o_ref[...] = jnp.where(rid < N_NODES, dinv * xw, 0.0)


def _final_body(p_ref, y_ref, dinv_ref, b_ref, batch_ref,
                wf1_ref, bf1_ref, wf2_ref, bf2_ref, o_ref):
  s = p_ref[0] + p_ref[1] + y_ref[...]
  h = jnp.maximum(dinv_ref[...] * s + b_ref[...], 0.0)
  hn = h[0:N_NODES, 0:64]
  gid = lax.broadcasted_iota(jnp.int32, (N_GRAPHS, N_NODES), 0)
  mask = (batch_ref[...] == gid).astype(jnp.float32)
  sums = _dot(mask, hn)
  cnt = jnp.sum(mask, axis=1, keepdims=True)
  pooled = sums / jnp.maximum(cnt, 1.0)
  hf = jnp.maximum(_dot(pooled, wf1_ref[...]) + bf1_ref[...], 0.0)
  o_ref[...] = _dot(hf, wf2_ref[...]) + bf2_ref[...]


def kernel(x, edge_index, batch, W1, b1, W2, b2, W3, b3, Wf1, bf1, Wf2, bf2):
  e = edge_index.shape[1]
  per_tile = -(-e // (NC * NS))
  cpt = -(-per_tile // CHUNK)
  cpt += cpt % 2
  e_pad = NC * NS * cpt * CHUNK

  row = edge_index[0].astype(jnp.int32)
  col = edge_index[1].astype(jnp.int32)
  fill = jnp.full((e_pad - e,), N_NODES, jnp.int32)
  rows = jnp.concatenate([row, fill]).reshape(NC, NS, cpt, CHUNK)
  cols = jnp.concatenate([col, fill]).reshape(NC, NS, cpt, CHUNK)

  ones128 = jnp.ones((DEG_CHUNK, 128), jnp.float32)
  zeros128 = jnp.zeros((N_PAD, 128), jnp.float32)

  W3p = jnp.pad(W3, ((0, 0), (0, 128 - W3.shape[1])))
  b3p = jnp.pad(b3, (0, 128 - b3.shape[0]))

  degp = _degree_sc(cols, ones128, zeros128)
  xw1 = _tc_matmul(x, W1)

  y1, dinv = pl.pallas_call(
      _prep_body,
      out_shape=(jax.ShapeDtypeStruct((N_PAD, 128), jnp.float32),
                 jax.ShapeDtypeStruct((N_PAD, 1), jnp.float32)),
  )(degp, xw1)

  p1 = _aggregate_sc(rows, cols, y1, zeros128)
  y2 = pl.pallas_call(
      _layer_body,
      out_shape=jax.ShapeDtypeStruct((N_PAD, 128), jnp.float32),
  )(p1, y1, dinv, b1.reshape(1, -1), W2)

  p2 = _aggregate_sc(rows, cols, y2, zeros128)
  y3 = pl.pallas_call(
      _layer_body,
      out_shape=jax.ShapeDtypeStruct((N_PAD, 128), jnp.float32),
  )(p2, y2, dinv, b2.reshape(1, -1), W3p)

  p3 = _aggregate_sc(rows, cols, y3, zeros128)
  out = pl.pallas_call(
      _final_body,
      out_shape=jax.ShapeDtypeStruct((N_GRAPHS, Wf2.shape[1]), jnp.float32),
  )(p3, y3, dinv, b3p.reshape(1, -1),
    batch.astype(jnp.int32).reshape(1, -1),
    Wf1, bf1.reshape(1, -1), Wf2, bf2.reshape(1, -1))
  return out
